# baseline (speedup 1.0000x reference)
import jax
import jax.numpy as jnp
from jax.experimental import pallas as pl
from jax.experimental.pallas import tpu as pltpu
from jax.experimental.pallas import tpu_sc as plsc

K = 32


def _sc_gather(data, idx2d, window=128):
    num = idx2d.shape[1]
    d = data.shape[1]
    mesh = plsc.VectorSubcoreMesh(core_axis_name="c", subcore_axis_name="s")

    @pl.kernel(
        out_type=jax.ShapeDtypeStruct((num, d), data.dtype),
        mesh=mesh,
    )
    def gather_kernel(data_hbm, i_hbm, o_hbm):
        def body(i_vmem, o_vmem):
            pltpu.sync_copy(data_hbm.at[i_vmem.at[0]], o_vmem)

        pltpu.emit_pipeline(
            body,
            grid=(num // window,),
            in_specs=[pl.BlockSpec((1, window), lambda i: (0, i))],
            out_specs=[pl.BlockSpec((window, d), lambda i: (i, 0))],
            core_axis_name=("c", "s"),
            dimension_semantics=(pltpu.PARALLEL,),
        )(i_hbm, o_hbm)

    return gather_kernel(data, idx2d)


def _rne_bf16(x):
    bits = jax.lax.bitcast_convert_type(x, jnp.uint32)
    rounded = ((bits + jnp.uint32(0x7FFF) + ((bits >> 16) & jnp.uint32(1)))
               & jnp.uint32(0xFFFF0000))
    return jax.lax.bitcast_convert_type(rounded, jnp.float32)


def _topk_body(xt_ref, q_ref, idx_ref, dist_ref):
    xt = xt_ref[0]
    q = q_ref[0]
    n = xt.shape[1]
    x0, x1, x2 = xt[0:1, :], xt[1:2, :], xt[2:3, :]
    q0, q1, q2 = q[:, 0:1], q[:, 1:2], q[:, 2:3]
    xr0, xr1, xr2 = _rne_bf16(x0), _rne_bf16(x1), _rne_bf16(x2)
    qr0, qr1, qr2 = _rne_bf16(q0), _rne_bf16(q1), _rne_bf16(q2)
    mm = qr0 * xr0 + qr1 * xr1 + qr2 * xr2
    qs = q0 * q0 + q1 * q1 + q2 * q2
    xs = x0 * x0 + x1 * x1 + x2 * x2
    dist_ref[...] = -2.0 * mm + qs + xs
    ch = 1024
    nc = n // ch
    ioc = jax.lax.broadcasted_iota(jnp.int32, (8, ch), 1)
    kio = jax.lax.broadcasted_iota(jnp.int32, (8, K), 1)
    inf = jnp.float32(jnp.inf)

    sv_parts, si_parts = [], []
    for c in range(nc):
        v = dist_ref[:, c * ch:(c + 1) * ch]

        def cbody(k, carry, v=v):
            mprev, aprev, accv, acci = carry
            sel = (v > mprev) | ((v == mprev) & (ioc > aprev))
            v2 = jnp.where(sel, v, inf)
            m = jnp.min(v2, axis=1, keepdims=True)
            a = jnp.min(jnp.where(v2 == m, ioc, jnp.int32(ch)),
                        axis=1, keepdims=True)
            accv = jnp.where(kio == k, m, accv)
            acci = jnp.where(kio == k, a, acci)
            return m, a, accv, acci

        _, _, accv, acci = jax.lax.fori_loop(
            0, K, cbody,
            (jnp.full((8, 1), -jnp.inf, jnp.float32),
             jnp.full((8, 1), -1, jnp.int32),
             jnp.zeros((8, K), jnp.float32),
             jnp.zeros((8, K), jnp.int32)))
        sv_parts.append(accv)
        si_parts.append(acci + jnp.int32(c * ch))
    sv = jnp.concatenate(sv_parts, axis=1)
    si = jnp.concatenate(si_parts, axis=1)

    def mbody(k, carry):
        mprev, gprev, acc = carry
        sel = (sv > mprev) | ((sv == mprev) & (si > gprev))
        v2 = jnp.where(sel, sv, inf)
        m = jnp.min(v2, axis=1, keepdims=True)
        g = jnp.min(jnp.where(v2 == m, si, jnp.int32(n)),
                    axis=1, keepdims=True)
        acc = jnp.where(kio == k, g, acc)
        return m, g, acc

    _, _, acc = jax.lax.fori_loop(
        0, K, mbody,
        (jnp.full((8, 1), -jnp.inf, jnp.float32),
         jnp.full((8, 1), -1, jnp.int32),
         jnp.zeros((8, K), jnp.int32)))
    idx_ref[0] = acc


def _dist_topk(xyz_t, new_xyz):
    b, _, n = xyz_t.shape
    g = new_xyz.shape[1]
    return pl.pallas_call(
        _topk_body,
        grid=(b, g // 8),
        in_specs=[
            pl.BlockSpec((1, 3, n), lambda i, j: (i, 0, 0)),
            pl.BlockSpec((1, 8, 3), lambda i, j: (i, j, 0)),
        ],
        out_specs=pl.BlockSpec((1, 8, K), lambda i, j: (i, j, 0)),
        out_shape=jax.ShapeDtypeStruct((b, g, K), jnp.int32),
        scratch_shapes=[pltpu.VMEM((8, n), jnp.float32)],
    )(xyz_t, new_xyz)


def _stats_body(x_ref, mean_ref, acc_ref):
    x = x_ref[...]
    s1 = jnp.sum(x, axis=1, keepdims=True)
    mean = s1 / jnp.float32(K)
    s2 = jnp.sum(x * x, axis=1, keepdims=True)
    part = jnp.sum(s2 - jnp.float32(K) * mean * mean).reshape(1, 1, 1)
    mean_ref[...] = mean

    @pl.when(pl.program_id(1) == 0)
    def _():
        acc_ref[...] = jnp.zeros_like(acc_ref)

    acc_ref[...] += part


def _stats(grouped3, b):
    bg, _, d = grouped3.shape
    g = bg // b
    gt = 64
    ng = g // gt
    return pl.pallas_call(
        _stats_body,
        grid=(b, ng),
        in_specs=[pl.BlockSpec((gt, K, d), lambda i, j: (i * ng + j, 0, 0))],
        out_specs=[
            pl.BlockSpec((gt, 1, d), lambda i, j: (i * ng + j, 0, 0)),
            pl.BlockSpec((1, 1, 1), lambda i, j: (i, 0, 0)),
        ],
        out_shape=[
            jax.ShapeDtypeStruct((bg, 1, d), jnp.float32),
            jax.ShapeDtypeStruct((b, 1, 1), jnp.float32),
        ],
    )(grouped3)


def _make_norm_body(nm1, d):
    def _norm_body(x_ref, mean_ref, sq_ref, ctr_ref, a_ref, b_ref, out_ref):
        x = x_ref[...]
        mean = mean_ref[...]
        std = jnp.sqrt(sq_ref[...] / jnp.float32(nm1))
        y = (x - mean) / (std + 1e-05)
        a = a_ref[...][:, None, :]
        bb = b_ref[...][:, None, :]
        y = a * y + bb
        ctr = ctr_ref[...]
        gt = x.shape[0]
        c = ctr.shape[2]
        ctr_b = jnp.broadcast_to(ctr, (gt, K, c))
        out_ref[...] = jnp.concatenate([y[:, :, :d], ctr_b], axis=2)
    return _norm_body


def _normalize(grouped3, mean, batchsq, ctr, alpha, beta, b, d):
    bg, _, dp = grouped3.shape
    c = ctr.shape[2]
    g = bg // b
    gt = 32
    ng = g // gt
    nm1 = g * K * d - 1
    return pl.pallas_call(
        _make_norm_body(nm1, d),
        grid=(b, ng),
        in_specs=[
            pl.BlockSpec((gt, K, dp), lambda i, j: (i * ng + j, 0, 0)),
            pl.BlockSpec((gt, 1, dp), lambda i, j: (i * ng + j, 0, 0)),
            pl.BlockSpec((1, 1, 1), lambda i, j: (i, 0, 0)),
            pl.BlockSpec((gt, 1, c), lambda i, j: (i * ng + j, 0, 0)),
            pl.BlockSpec((1, dp), lambda i, j: (0, 0)),
            pl.BlockSpec((1, dp), lambda i, j: (0, 0)),
        ],
        out_specs=pl.BlockSpec((gt, K, d + c), lambda i, j: (i * ng + j, 0, 0)),
        out_shape=jax.ShapeDtypeStruct((bg, K, d + c), jnp.float32),
    )(grouped3, mean, batchsq, ctr, alpha, beta)


def kernel(xyz, points, fps_idx, affine_alpha, affine_beta):
    b, n, c = points.shape
    g = fps_idx.shape[1]
    d = c + 3
    dp = 256

    comb = jnp.concatenate(
        [points, xyz, jnp.zeros((b, n, dp - d), jnp.float32)], axis=-1
    ).reshape(b * n, dp)
    fps32 = fps_idx.astype(jnp.int32)
    boff = (jnp.arange(b, dtype=jnp.int32) * n)
    fps_flat = (fps32 + boff[:, None]).reshape(1, b * g)

    newc = _sc_gather(comb, fps_flat)
    new_xyz = newc[:, c:d].reshape(b, g, 3)

    idx = _dist_topk(xyz.transpose(0, 2, 1), new_xyz)
    idx_flat = (idx + boff[:, None, None]).reshape(1, b * g * K)

    grouped = _sc_gather(comb, idx_flat)
    grouped3 = grouped.reshape(b * g, K, dp)

    mean, batchsq = _stats(grouped3, b)
    ctr = newc[:, :c].reshape(b * g, 1, c)
    alpha_p = jnp.pad(affine_alpha.reshape(1, d), ((0, 0), (0, dp - d)))
    beta_p = jnp.pad(affine_beta.reshape(1, d), ((0, 0), (0, dp - d)))
    out = _normalize(grouped3, mean, batchsq, ctr, alpha_p, beta_p, b, d)
    return new_xyz, out.reshape(b, g, K, 2 * c + 3)


# speedup vs baseline: 7.8377x; 7.8377x over previous
import jax
import jax.numpy as jnp
from jax.experimental import pallas as pl
from jax.experimental.pallas import tpu as pltpu
from jax.experimental.pallas import tpu_sc as plsc

K = 32


def _sc_gather(data, idx2d, window=128):
    num = idx2d.shape[1]
    d = data.shape[1]
    mesh = plsc.VectorSubcoreMesh(core_axis_name="c", subcore_axis_name="s")

    @pl.kernel(
        out_type=jax.ShapeDtypeStruct((num, d), data.dtype),
        mesh=mesh,
    )
    def gather_kernel(data_hbm, i_hbm, o_hbm):
        def body(i_vmem, o_vmem):
            pltpu.sync_copy(data_hbm.at[i_vmem.at[0]], o_vmem)

        pltpu.emit_pipeline(
            body,
            grid=(num // window,),
            in_specs=[pl.BlockSpec((1, window), lambda i: (0, i))],
            out_specs=[pl.BlockSpec((window, d), lambda i: (i, 0))],
            core_axis_name=("c", "s"),
            dimension_semantics=(pltpu.PARALLEL,),
        )(i_hbm, o_hbm)

    return gather_kernel(data, idx2d)


def _rne_bf16(x):
    bits = jax.lax.bitcast_convert_type(x, jnp.uint32)
    rounded = ((bits + jnp.uint32(0x7FFF) + ((bits >> 16) & jnp.uint32(1)))
               & jnp.uint32(0xFFFF0000))
    return jax.lax.bitcast_convert_type(rounded, jnp.float32)


_SLABS = 4


def _topk_body(xt_ref, q_ref, idx_ref, dist_ref):
    xt = xt_ref[0]
    q = q_ref[0]
    n = xt.shape[1]
    rows = 8 * _SLABS
    x0, x1, x2 = xt[0:1, :], xt[1:2, :], xt[2:3, :]
    q0, q1, q2 = q[:, 0:1], q[:, 1:2], q[:, 2:3]
    xr0, xr1, xr2 = _rne_bf16(x0), _rne_bf16(x1), _rne_bf16(x2)
    qr0, qr1, qr2 = _rne_bf16(q0), _rne_bf16(q1), _rne_bf16(q2)
    mm = qr0 * xr0 + qr1 * xr1 + qr2 * xr2
    qs = q0 * q0 + q1 * q1 + q2 * q2
    xs = x0 * x0 + x1 * x1 + x2 * x2
    dist_ref[...] = -2.0 * mm + qs + xs
    ch = 512
    nc = n // ch
    ioc = jax.lax.broadcasted_iota(jnp.int32, (8, ch), 1)
    kio = jax.lax.broadcasted_iota(jnp.int32, (8, K), 1)
    inf = jnp.float32(jnp.inf)
    big = jnp.int32(n)

    def body(k, carry):
        out = []
        for s in range(_SLABS):
            mp, ap, acc = carry[s]
            rv = jnp.full((8, ch), inf, jnp.float32)
            ri = jnp.full((8, ch), big, jnp.int32)
            for c in range(nc):
                v = dist_ref[8 * s:8 * s + 8, ch * c:ch * (c + 1)]
                gi = ioc + jnp.int32(ch * c)
                valid = (v > mp) | ((v == mp) & (gi > ap))
                v2 = jnp.where(valid, v, inf)
                upd = (v2 < rv) | ((v2 == rv) & (gi < ri))
                rv = jnp.where(upd, v2, rv)
                ri = jnp.where(upd, gi, ri)
            m = jnp.min(rv, axis=1, keepdims=True)
            g = jnp.min(jnp.where(rv == m, ri, big), axis=1, keepdims=True)
            acc = jnp.where(kio == k, g, acc)
            out.append((m, g, acc))
        return tuple(out)

    init = tuple(
        (jnp.full((8, 1), -jnp.inf, jnp.float32),
         jnp.full((8, 1), -1, jnp.int32),
         jnp.zeros((8, K), jnp.int32))
        for _ in range(_SLABS))
    fin = jax.lax.fori_loop(0, K, body, init)
    idx_ref[0] = jnp.concatenate([fin[s][2] for s in range(_SLABS)], axis=0)


def _dist_topk(xyz_t, new_xyz):
    b, _, n = xyz_t.shape
    g = new_xyz.shape[1]
    rows = 8 * _SLABS
    return pl.pallas_call(
        _topk_body,
        grid=(b, g // rows),
        in_specs=[
            pl.BlockSpec((1, 3, n), lambda i, j: (i, 0, 0)),
            pl.BlockSpec((1, rows, 3), lambda i, j: (i, j, 0)),
        ],
        out_specs=pl.BlockSpec((1, rows, K), lambda i, j: (i, j, 0)),
        out_shape=jax.ShapeDtypeStruct((b, g, K), jnp.int32),
        scratch_shapes=[pltpu.VMEM((rows, n), jnp.float32)],
    )(xyz_t, new_xyz)


def _stats_body(x_ref, mean_ref, acc_ref):
    x = x_ref[...]
    s1 = jnp.sum(x, axis=1, keepdims=True)
    mean = s1 / jnp.float32(K)
    s2 = jnp.sum(x * x, axis=1, keepdims=True)
    part = jnp.sum(s2 - jnp.float32(K) * mean * mean).reshape(1, 1, 1)
    mean_ref[...] = mean

    @pl.when(pl.program_id(1) == 0)
    def _():
        acc_ref[...] = jnp.zeros_like(acc_ref)

    acc_ref[...] += part


def _stats(grouped3, b):
    bg, _, d = grouped3.shape
    g = bg // b
    gt = 64
    ng = g // gt
    return pl.pallas_call(
        _stats_body,
        grid=(b, ng),
        in_specs=[pl.BlockSpec((gt, K, d), lambda i, j: (i * ng + j, 0, 0))],
        out_specs=[
            pl.BlockSpec((gt, 1, d), lambda i, j: (i * ng + j, 0, 0)),
            pl.BlockSpec((1, 1, 1), lambda i, j: (i, 0, 0)),
        ],
        out_shape=[
            jax.ShapeDtypeStruct((bg, 1, d), jnp.float32),
            jax.ShapeDtypeStruct((b, 1, 1), jnp.float32),
        ],
    )(grouped3)


def _make_norm_body(nm1, d):
    def _norm_body(x_ref, mean_ref, sq_ref, ctr_ref, a_ref, b_ref, out_ref):
        x = x_ref[...]
        mean = mean_ref[...]
        std = jnp.sqrt(sq_ref[...] / jnp.float32(nm1))
        y = (x - mean) / (std + 1e-05)
        a = a_ref[...][:, None, :]
        bb = b_ref[...][:, None, :]
        y = a * y + bb
        ctr = ctr_ref[...]
        gt = x.shape[0]
        c = ctr.shape[2]
        ctr_b = jnp.broadcast_to(ctr, (gt, K, c))
        out_ref[...] = jnp.concatenate([y[:, :, :d], ctr_b], axis=2)
    return _norm_body


def _normalize(grouped3, mean, batchsq, ctr, alpha, beta, b, d):
    bg, _, dp = grouped3.shape
    c = ctr.shape[2]
    g = bg // b
    gt = 32
    ng = g // gt
    nm1 = g * K * d - 1
    return pl.pallas_call(
        _make_norm_body(nm1, d),
        grid=(b, ng),
        in_specs=[
            pl.BlockSpec((gt, K, dp), lambda i, j: (i * ng + j, 0, 0)),
            pl.BlockSpec((gt, 1, dp), lambda i, j: (i * ng + j, 0, 0)),
            pl.BlockSpec((1, 1, 1), lambda i, j: (i, 0, 0)),
            pl.BlockSpec((gt, 1, c), lambda i, j: (i * ng + j, 0, 0)),
            pl.BlockSpec((1, dp), lambda i, j: (0, 0)),
            pl.BlockSpec((1, dp), lambda i, j: (0, 0)),
        ],
        out_specs=pl.BlockSpec((gt, K, d + c), lambda i, j: (i * ng + j, 0, 0)),
        out_shape=jax.ShapeDtypeStruct((bg, K, d + c), jnp.float32),
    )(grouped3, mean, batchsq, ctr, alpha, beta)


def kernel(xyz, points, fps_idx, affine_alpha, affine_beta):
    b, n, c = points.shape
    g = fps_idx.shape[1]
    d = c + 3
    dp = 256

    comb = jnp.concatenate(
        [points, xyz, jnp.zeros((b, n, dp - d), jnp.float32)], axis=-1
    ).reshape(b * n, dp)
    fps32 = fps_idx.astype(jnp.int32)
    boff = (jnp.arange(b, dtype=jnp.int32) * n)
    fps_flat = (fps32 + boff[:, None]).reshape(1, b * g)

    newc = _sc_gather(comb, fps_flat)
    new_xyz = newc[:, c:d].reshape(b, g, 3)

    idx = _dist_topk(xyz.transpose(0, 2, 1), new_xyz)
    idx_flat = (idx + boff[:, None, None]).reshape(1, b * g * K)

    grouped = _sc_gather(comb, idx_flat)
    grouped3 = grouped.reshape(b * g, K, dp)

    mean, batchsq = _stats(grouped3, b)
    ctr = newc[:, :c].reshape(b * g, 1, c)
    alpha_p = jnp.pad(affine_alpha.reshape(1, d), ((0, 0), (0, dp - d)))
    beta_p = jnp.pad(affine_beta.reshape(1, d), ((0, 0), (0, dp - d)))
    out = _normalize(grouped3, mean, batchsq, ctr, alpha_p, beta_p, b, d)
    return new_xyz, out.reshape(b, g, K, 2 * c + 3)


# speedup vs baseline: 9.7131x; 1.2393x over previous
import jax
import jax.numpy as jnp
from jax.experimental import pallas as pl
from jax.experimental.pallas import tpu as pltpu
from jax.experimental.pallas import tpu_sc as plsc

K = 32


def _sc_gather(data, idx2d, window=128):
    num = idx2d.shape[1]
    d = data.shape[1]
    mesh = plsc.VectorSubcoreMesh(core_axis_name="c", subcore_axis_name="s")

    @pl.kernel(
        out_type=jax.ShapeDtypeStruct((num, d), data.dtype),
        mesh=mesh,
    )
    def gather_kernel(data_hbm, i_hbm, o_hbm):
        def body(i_vmem, o_vmem):
            pltpu.sync_copy(data_hbm.at[i_vmem.at[0]], o_vmem)

        pltpu.emit_pipeline(
            body,
            grid=(num // window,),
            in_specs=[pl.BlockSpec((1, window), lambda i: (0, i))],
            out_specs=[pl.BlockSpec((window, d), lambda i: (i, 0))],
            core_axis_name=("c", "s"),
            dimension_semantics=(pltpu.PARALLEL,),
        )(i_hbm, o_hbm)

    return gather_kernel(data, idx2d)


def _rne_bf16(x):
    bits = jax.lax.bitcast_convert_type(x, jnp.uint32)
    rounded = ((bits + jnp.uint32(0x7FFF) + ((bits >> 16) & jnp.uint32(1)))
               & jnp.uint32(0xFFFF0000))
    return jax.lax.bitcast_convert_type(rounded, jnp.float32)


_SLABS = 8


def _topk_body(xt_ref, q_ref, idx_ref, dist_ref):
    xt = xt_ref[0]
    q = q_ref[0]
    n = xt.shape[1]
    rows = 8 * _SLABS
    x0, x1, x2 = xt[0:1, :], xt[1:2, :], xt[2:3, :]
    q0, q1, q2 = q[:, 0:1], q[:, 1:2], q[:, 2:3]
    xr0, xr1, xr2 = _rne_bf16(x0), _rne_bf16(x1), _rne_bf16(x2)
    qr0, qr1, qr2 = _rne_bf16(q0), _rne_bf16(q1), _rne_bf16(q2)
    mm = qr0 * xr0 + qr1 * xr1 + qr2 * xr2
    qs = q0 * q0 + q1 * q1 + q2 * q2
    xs = x0 * x0 + x1 * x1 + x2 * x2
    dist_ref[...] = -2.0 * mm + qs + xs
    ch = 512
    nc = n // ch
    ioc = jax.lax.broadcasted_iota(jnp.int32, (8, ch), 1)
    kio = jax.lax.broadcasted_iota(jnp.int32, (8, K), 1)
    inf = jnp.float32(jnp.inf)
    big = jnp.int32(n)

    def body(k, carry):
        out = []
        for s in range(_SLABS):
            mp, ap, acc = carry[s]
            rv = jnp.full((8, ch), inf, jnp.float32)
            ri = jnp.full((8, ch), big, jnp.int32)
            for c in range(nc):
                v = dist_ref[8 * s:8 * s + 8, ch * c:ch * (c + 1)]
                gi = ioc + jnp.int32(ch * c)
                valid = (v > mp) | ((v == mp) & (gi > ap))
                v2 = jnp.where(valid, v, inf)
                upd = (v2 < rv) | ((v2 == rv) & (gi < ri))
                rv = jnp.where(upd, v2, rv)
                ri = jnp.where(upd, gi, ri)
            m = jnp.min(rv, axis=1, keepdims=True)
            g = jnp.min(jnp.where(rv == m, ri, big), axis=1, keepdims=True)
            acc = jnp.where(kio == k, g, acc)
            out.append((m, g, acc))
        return tuple(out)

    init = tuple(
        (jnp.full((8, 1), -jnp.inf, jnp.float32),
         jnp.full((8, 1), -1, jnp.int32),
         jnp.zeros((8, K), jnp.int32))
        for _ in range(_SLABS))
    fin = jax.lax.fori_loop(0, K, body, init)
    idx_ref[0] = jnp.concatenate([fin[s][2] for s in range(_SLABS)], axis=0)


def _dist_topk(xyz_t, new_xyz):
    b, _, n = xyz_t.shape
    g = new_xyz.shape[1]
    rows = 8 * _SLABS
    return pl.pallas_call(
        _topk_body,
        grid=(b, g // rows),
        in_specs=[
            pl.BlockSpec((1, 3, n), lambda i, j: (i, 0, 0)),
            pl.BlockSpec((1, rows, 3), lambda i, j: (i, j, 0)),
        ],
        out_specs=pl.BlockSpec((1, rows, K), lambda i, j: (i, j, 0)),
        out_shape=jax.ShapeDtypeStruct((b, g, K), jnp.int32),
        scratch_shapes=[pltpu.VMEM((rows, n), jnp.float32)],
    )(xyz_t, new_xyz)


def _stats_body(x_ref, mean_ref, acc_ref):
    x = x_ref[...]
    s1 = jnp.sum(x, axis=1, keepdims=True)
    mean = s1 / jnp.float32(K)
    s2 = jnp.sum(x * x, axis=1, keepdims=True)
    part = jnp.sum(s2 - jnp.float32(K) * mean * mean).reshape(1, 1, 1)
    mean_ref[...] = mean

    @pl.when(pl.program_id(1) == 0)
    def _():
        acc_ref[...] = jnp.zeros_like(acc_ref)

    acc_ref[...] += part


def _stats(grouped3, b):
    bg, _, d = grouped3.shape
    g = bg // b
    gt = 64
    ng = g // gt
    return pl.pallas_call(
        _stats_body,
        grid=(b, ng),
        in_specs=[pl.BlockSpec((gt, K, d), lambda i, j: (i * ng + j, 0, 0))],
        out_specs=[
            pl.BlockSpec((gt, 1, d), lambda i, j: (i * ng + j, 0, 0)),
            pl.BlockSpec((1, 1, 1), lambda i, j: (i, 0, 0)),
        ],
        out_shape=[
            jax.ShapeDtypeStruct((bg, 1, d), jnp.float32),
            jax.ShapeDtypeStruct((b, 1, 1), jnp.float32),
        ],
    )(grouped3)


def _make_norm_body(nm1, d):
    def _norm_body(x_ref, mean_ref, sq_ref, ctr_ref, a_ref, b_ref, out_ref):
        x = x_ref[...]
        mean = mean_ref[...]
        std = jnp.sqrt(sq_ref[...] / jnp.float32(nm1))
        y = (x - mean) / (std + 1e-05)
        a = a_ref[...][:, None, :]
        bb = b_ref[...][:, None, :]
        y = a * y + bb
        ctr = ctr_ref[...]
        gt = x.shape[0]
        c = ctr.shape[2]
        ctr_b = jnp.broadcast_to(ctr, (gt, K, c))
        out_ref[...] = jnp.concatenate([y[:, :, :d], ctr_b], axis=2)
    return _norm_body


def _normalize(grouped3, mean, batchsq, ctr, alpha, beta, b, d):
    bg, _, dp = grouped3.shape
    c = ctr.shape[2]
    g = bg // b
    gt = 32
    ng = g // gt
    nm1 = g * K * d - 1
    return pl.pallas_call(
        _make_norm_body(nm1, d),
        grid=(b, ng),
        in_specs=[
            pl.BlockSpec((gt, K, dp), lambda i, j: (i * ng + j, 0, 0)),
            pl.BlockSpec((gt, 1, dp), lambda i, j: (i * ng + j, 0, 0)),
            pl.BlockSpec((1, 1, 1), lambda i, j: (i, 0, 0)),
            pl.BlockSpec((gt, 1, c), lambda i, j: (i * ng + j, 0, 0)),
            pl.BlockSpec((1, dp), lambda i, j: (0, 0)),
            pl.BlockSpec((1, dp), lambda i, j: (0, 0)),
        ],
        out_specs=pl.BlockSpec((gt, K, d + c), lambda i, j: (i * ng + j, 0, 0)),
        out_shape=jax.ShapeDtypeStruct((bg, K, d + c), jnp.float32),
    )(grouped3, mean, batchsq, ctr, alpha, beta)


def kernel(xyz, points, fps_idx, affine_alpha, affine_beta):
    b, n, c = points.shape
    g = fps_idx.shape[1]
    d = c + 3
    dp = 256

    comb = jnp.concatenate(
        [points, xyz, jnp.zeros((b, n, dp - d), jnp.float32)], axis=-1
    ).reshape(b * n, dp)
    fps32 = fps_idx.astype(jnp.int32)
    boff = (jnp.arange(b, dtype=jnp.int32) * n)
    fps_flat = (fps32 + boff[:, None]).reshape(1, b * g)

    newc = _sc_gather(comb, fps_flat)
    new_xyz = newc[:, c:d].reshape(b, g, 3)

    idx = _dist_topk(xyz.transpose(0, 2, 1), new_xyz)
    idx_flat = (idx + boff[:, None, None]).reshape(1, b * g * K)

    grouped = _sc_gather(comb, idx_flat)
    grouped3 = grouped.reshape(b * g, K, dp)

    mean, batchsq = _stats(grouped3, b)
    ctr = newc[:, :c].reshape(b * g, 1, c)
    alpha_p = jnp.pad(affine_alpha.reshape(1, d), ((0, 0), (0, dp - d)))
    beta_p = jnp.pad(affine_beta.reshape(1, d), ((0, 0), (0, dp - d)))
    out = _normalize(grouped3, mean, batchsq, ctr, alpha_p, beta_p, b, d)
    return new_xyz, out.reshape(b, g, K, 2 * c + 3)


# speedup vs baseline: 11.5920x; 1.1934x over previous
import jax
import jax.numpy as jnp
from jax.experimental import pallas as pl
from jax.experimental.pallas import tpu as pltpu
from jax.experimental.pallas import tpu_sc as plsc

K = 32


def _sc_gather(data, idx2d, window=128):
    num = idx2d.shape[1]
    d = data.shape[1]
    mesh = plsc.VectorSubcoreMesh(core_axis_name="c", subcore_axis_name="s")

    @pl.kernel(
        out_type=jax.ShapeDtypeStruct((num, d), data.dtype),
        mesh=mesh,
    )
    def gather_kernel(data_hbm, i_hbm, o_hbm):
        def body(i_vmem, o_vmem):
            pltpu.sync_copy(data_hbm.at[i_vmem.at[0]], o_vmem)

        pltpu.emit_pipeline(
            body,
            grid=(num // window,),
            in_specs=[pl.BlockSpec((1, window), lambda i: (0, i))],
            out_specs=[pl.BlockSpec((window, d), lambda i: (i, 0))],
            core_axis_name=("c", "s"),
            dimension_semantics=(pltpu.PARALLEL,),
        )(i_hbm, o_hbm)

    return gather_kernel(data, idx2d)


def _rne_bf16(x):
    bits = jax.lax.bitcast_convert_type(x, jnp.uint32)
    rounded = ((bits + jnp.uint32(0x7FFF) + ((bits >> 16) & jnp.uint32(1)))
               & jnp.uint32(0xFFFF0000))
    return jax.lax.bitcast_convert_type(rounded, jnp.float32)


_SLABS = 32


def _topk_body(xt_ref, q_ref, idx_ref, dist_ref):
    xt = xt_ref[0]
    q = q_ref[0]
    n = xt.shape[1]
    rows = 8 * _SLABS
    x0, x1, x2 = xt[0:1, :], xt[1:2, :], xt[2:3, :]
    q0, q1, q2 = q[:, 0:1], q[:, 1:2], q[:, 2:3]
    xr0, xr1, xr2 = _rne_bf16(x0), _rne_bf16(x1), _rne_bf16(x2)
    qr0, qr1, qr2 = _rne_bf16(q0), _rne_bf16(q1), _rne_bf16(q2)
    mm = qr0 * xr0 + qr1 * xr1 + qr2 * xr2
    qs = q0 * q0 + q1 * q1 + q2 * q2
    xs = x0 * x0 + x1 * x1 + x2 * x2
    dist_ref[...] = -2.0 * mm + qs + xs
    ch = 512
    nc = n // ch
    ioc = jax.lax.broadcasted_iota(jnp.int32, (8, ch), 1)
    kio = jax.lax.broadcasted_iota(jnp.int32, (8, K), 1)
    inf = jnp.float32(jnp.inf)
    big = jnp.int32(n)

    def body(k, carry):
        out = []
        for s in range(_SLABS):
            mp, ap, acc = carry[s]
            rv = jnp.full((8, ch), inf, jnp.float32)
            ri = jnp.full((8, ch), big, jnp.int32)
            for c in range(nc):
                v = dist_ref[8 * s:8 * s + 8, ch * c:ch * (c + 1)]
                gi = ioc + jnp.int32(ch * c)
                valid = (v > mp) | ((v == mp) & (gi > ap))
                v2 = jnp.where(valid, v, inf)
                upd = (v2 < rv) | ((v2 == rv) & (gi < ri))
                rv = jnp.where(upd, v2, rv)
                ri = jnp.where(upd, gi, ri)
            m = jnp.min(rv, axis=1, keepdims=True)
            g = jnp.min(jnp.where(rv == m, ri, big), axis=1, keepdims=True)
            acc = jnp.where(kio == k, g, acc)
            out.append((m, g, acc))
        return tuple(out)

    init = tuple(
        (jnp.full((8, 1), -jnp.inf, jnp.float32),
         jnp.full((8, 1), -1, jnp.int32),
         jnp.zeros((8, K), jnp.int32))
        for _ in range(_SLABS))
    fin = jax.lax.fori_loop(0, K, body, init)
    idx_ref[0] = jnp.concatenate([fin[s][2] for s in range(_SLABS)], axis=0)


def _dist_topk(xyz_t, new_xyz):
    b, _, n = xyz_t.shape
    g = new_xyz.shape[1]
    rows = 8 * _SLABS
    return pl.pallas_call(
        _topk_body,
        grid=(b, g // rows),
        in_specs=[
            pl.BlockSpec((1, 3, n), lambda i, j: (i, 0, 0)),
            pl.BlockSpec((1, rows, 3), lambda i, j: (i, j, 0)),
        ],
        out_specs=pl.BlockSpec((1, rows, K), lambda i, j: (i, j, 0)),
        out_shape=jax.ShapeDtypeStruct((b, g, K), jnp.int32),
        scratch_shapes=[pltpu.VMEM((rows, n), jnp.float32)],
    )(xyz_t, new_xyz)


def _stats_body(x_ref, mean_ref, acc_ref):
    x = x_ref[...]
    s1 = jnp.sum(x, axis=1, keepdims=True)
    mean = s1 / jnp.float32(K)
    s2 = jnp.sum(x * x, axis=1, keepdims=True)
    part = jnp.sum(s2 - jnp.float32(K) * mean * mean).reshape(1, 1, 1)
    mean_ref[...] = mean

    @pl.when(pl.program_id(1) == 0)
    def _():
        acc_ref[...] = jnp.zeros_like(acc_ref)

    acc_ref[...] += part


def _stats(grouped3, b):
    bg, _, d = grouped3.shape
    g = bg // b
    gt = 64
    ng = g // gt
    return pl.pallas_call(
        _stats_body,
        grid=(b, ng),
        in_specs=[pl.BlockSpec((gt, K, d), lambda i, j: (i * ng + j, 0, 0))],
        out_specs=[
            pl.BlockSpec((gt, 1, d), lambda i, j: (i * ng + j, 0, 0)),
            pl.BlockSpec((1, 1, 1), lambda i, j: (i, 0, 0)),
        ],
        out_shape=[
            jax.ShapeDtypeStruct((bg, 1, d), jnp.float32),
            jax.ShapeDtypeStruct((b, 1, 1), jnp.float32),
        ],
    )(grouped3)


def _make_norm_body(nm1, d):
    def _norm_body(x_ref, mean_ref, sq_ref, ctr_ref, a_ref, b_ref, out_ref):
        x = x_ref[...]
        mean = mean_ref[...]
        std = jnp.sqrt(sq_ref[...] / jnp.float32(nm1))
        y = (x - mean) / (std + 1e-05)
        a = a_ref[...][:, None, :]
        bb = b_ref[...][:, None, :]
        y = a * y + bb
        ctr = ctr_ref[...]
        gt = x.shape[0]
        c = ctr.shape[2]
        ctr_b = jnp.broadcast_to(ctr, (gt, K, c))
        out_ref[...] = jnp.concatenate([y[:, :, :d], ctr_b], axis=2)
    return _norm_body


def _normalize(grouped3, mean, batchsq, ctr, alpha, beta, b, d):
    bg, _, dp = grouped3.shape
    c = ctr.shape[2]
    g = bg // b
    gt = 32
    ng = g // gt
    nm1 = g * K * d - 1
    return pl.pallas_call(
        _make_norm_body(nm1, d),
        grid=(b, ng),
        in_specs=[
            pl.BlockSpec((gt, K, dp), lambda i, j: (i * ng + j, 0, 0)),
            pl.BlockSpec((gt, 1, dp), lambda i, j: (i * ng + j, 0, 0)),
            pl.BlockSpec((1, 1, 1), lambda i, j: (i, 0, 0)),
            pl.BlockSpec((gt, 1, c), lambda i, j: (i * ng + j, 0, 0)),
            pl.BlockSpec((1, dp), lambda i, j: (0, 0)),
            pl.BlockSpec((1, dp), lambda i, j: (0, 0)),
        ],
        out_specs=pl.BlockSpec((gt, K, d + c), lambda i, j: (i * ng + j, 0, 0)),
        out_shape=jax.ShapeDtypeStruct((bg, K, d + c), jnp.float32),
    )(grouped3, mean, batchsq, ctr, alpha, beta)


def kernel(xyz, points, fps_idx, affine_alpha, affine_beta):
    b, n, c = points.shape
    g = fps_idx.shape[1]
    d = c + 3
    dp = 256

    comb = jnp.concatenate(
        [points, xyz, jnp.zeros((b, n, dp - d), jnp.float32)], axis=-1
    ).reshape(b * n, dp)
    fps32 = fps_idx.astype(jnp.int32)
    boff = (jnp.arange(b, dtype=jnp.int32) * n)
    fps_flat = (fps32 + boff[:, None]).reshape(1, b * g)

    newc = _sc_gather(comb, fps_flat)
    new_xyz = newc[:, c:d].reshape(b, g, 3)

    idx = _dist_topk(xyz.transpose(0, 2, 1), new_xyz)
    idx_flat = (idx + boff[:, None, None]).reshape(1, b * g * K)

    grouped = _sc_gather(comb, idx_flat)
    grouped3 = grouped.reshape(b * g, K, dp)

    mean, batchsq = _stats(grouped3, b)
    ctr = newc[:, :c].reshape(b * g, 1, c)
    alpha_p = jnp.pad(affine_alpha.reshape(1, d), ((0, 0), (0, dp - d)))
    beta_p = jnp.pad(affine_beta.reshape(1, d), ((0, 0), (0, dp - d)))
    out = _normalize(grouped3, mean, batchsq, ctr, alpha_p, beta_p, b, d)
    return new_xyz, out.reshape(b, g, K, 2 * c + 3)
